# Initial kernel scaffold; baseline (speedup 1.0000x reference)
import jax
import jax.numpy as jnp
from jax.experimental import pallas as pl


def kernel(routing_inputs, W):
    raise NotImplementedError("write your pallas kernel here")


# speedup vs baseline: 8.0492x; 8.0492x over previous
import jax
import jax.numpy as jnp
from jax.experimental import pallas as pl

_TOP_P = 0.8
_E = 64
_T_BLK = 512


def _gate_kernel(x_ref, w_ref, o_ref):
    x = x_ref[...]
    w = w_ref[...]
    logits = jax.lax.dot_general(
        x, w, (((1,), (1,)), ((), ())),
        preferred_element_type=jnp.float32,
    )
    m = jnp.max(logits, axis=-1, keepdims=True)
    ex = jnp.exp(logits - m)
    probs = ex / jnp.sum(ex, axis=-1, keepdims=True)

    col = jax.lax.broadcasted_iota(jnp.int32, probs.shape, 1)
    cols = []
    for e in range(_E):
        pe = probs[:, e:e + 1]
        above = (probs > pe) | ((probs == pe) & (col < e))
        s_e = jnp.sum(jnp.where(above, probs, 0.0), axis=-1, keepdims=True)
        cols.append(s_e)
    s = jnp.concatenate(cols, axis=-1)
    score = (1.0 + probs) - probs
    o_ref[...] = jnp.where(s <= _TOP_P, score, 0.0)


def kernel(routing_inputs, W):
    n_tok, hidden = routing_inputs.shape
    return pl.pallas_call(
        _gate_kernel,
        grid=(n_tok // _T_BLK,),
        in_specs=[
            pl.BlockSpec((_T_BLK, hidden), lambda i: (i, 0)),
            pl.BlockSpec((_E, hidden), lambda i: (0, 0)),
        ],
        out_specs=pl.BlockSpec((_T_BLK, _E), lambda i: (i, 0)),
        out_shape=jax.ShapeDtypeStruct((n_tok, _E), jnp.float32),
    )(routing_inputs, W)


# speedup vs baseline: 34.4746x; 4.2830x over previous
import jax
import jax.numpy as jnp
from jax.experimental import pallas as pl

_TOP_P = 0.8
_E = 64
_T_BLK = 512


def _gate_kernel(x_ref, w_ref, o_ref):
    x = x_ref[...]
    w = w_ref[...]
    logits_t = jax.lax.dot_general(
        w, x, (((1,), (1,)), ((), ())),
        preferred_element_type=jnp.float32,
    )
    m = jnp.max(logits_t, axis=0, keepdims=True)
    ex = jnp.exp(logits_t - m)
    probs = ex / jnp.sum(ex, axis=0, keepdims=True)

    row = jax.lax.broadcasted_iota(jnp.int32, probs.shape, 0)
    rows = []
    for e in range(_E):
        pe = probs[e:e + 1, :]
        above = (probs > pe) | ((probs == pe) & (row < e))
        s_e = jnp.sum(jnp.where(above, probs, 0.0), axis=0, keepdims=True)
        rows.append(s_e)
    s = jnp.concatenate(rows, axis=0)
    score = (1.0 + probs) - probs
    out_t = jnp.where(s <= _TOP_P, score, 0.0)
    eye = (jax.lax.broadcasted_iota(jnp.int32, (_E, _E), 0)
           == jax.lax.broadcasted_iota(jnp.int32, (_E, _E), 1)
           ).astype(jnp.float32)
    o_ref[...] = jax.lax.dot_general(
        out_t, eye, (((0,), (0,)), ((), ())),
        preferred_element_type=jnp.float32,
        precision=jax.lax.Precision.HIGHEST,
    )


def kernel(routing_inputs, W):
    n_tok, hidden = routing_inputs.shape
    return pl.pallas_call(
        _gate_kernel,
        grid=(n_tok // _T_BLK,),
        in_specs=[
            pl.BlockSpec((_T_BLK, hidden), lambda i: (i, 0)),
            pl.BlockSpec((_E, hidden), lambda i: (0, 0)),
        ],
        out_specs=pl.BlockSpec((_T_BLK, _E), lambda i: (i, 0)),
        out_shape=jax.ShapeDtypeStruct((n_tok, _E), jnp.float32),
    )(routing_inputs, W)


# speedup vs baseline: 39.7625x; 1.1534x over previous
import jax
import jax.numpy as jnp
from jax.experimental import pallas as pl

_TOP_P = 0.8
_E = 64
_T_BLK = 1024


def _gate_kernel(x_ref, w_ref, o_ref):
    x = x_ref[...]
    w = w_ref[...]
    logits_t = jax.lax.dot_general(
        w, x, (((1,), (1,)), ((), ())),
        preferred_element_type=jnp.float32,
    )
    m = jnp.max(logits_t, axis=0, keepdims=True)
    ex = jnp.exp(logits_t - m)
    probs = ex / jnp.sum(ex, axis=0, keepdims=True)

    row = jax.lax.broadcasted_iota(jnp.int32, probs.shape, 0)
    rows = []
    for e in range(_E):
        pe = probs[e:e + 1, :]
        above = (probs > pe) | ((probs == pe) & (row < e))
        s_e = jnp.sum(jnp.where(above, probs, 0.0), axis=0, keepdims=True)
        rows.append(s_e)
    s = jnp.concatenate(rows, axis=0)
    score = (1.0 + probs) - probs
    out_t = jnp.where(s <= _TOP_P, score, 0.0)
    eye = (jax.lax.broadcasted_iota(jnp.int32, (_E, _E), 0)
           == jax.lax.broadcasted_iota(jnp.int32, (_E, _E), 1)
           ).astype(jnp.float32)
    o_ref[...] = jax.lax.dot_general(
        out_t, eye, (((0,), (0,)), ((), ())),
        preferred_element_type=jnp.float32,
        precision=jax.lax.Precision.HIGHEST,
    )


def kernel(routing_inputs, W):
    n_tok, hidden = routing_inputs.shape
    return pl.pallas_call(
        _gate_kernel,
        grid=(n_tok // _T_BLK,),
        in_specs=[
            pl.BlockSpec((_T_BLK, hidden), lambda i: (i, 0)),
            pl.BlockSpec((_E, hidden), lambda i: (0, 0)),
        ],
        out_specs=pl.BlockSpec((_T_BLK, _E), lambda i: (i, 0)),
        out_shape=jax.ShapeDtypeStruct((n_tok, _E), jnp.float32),
    )(routing_inputs, W)


# speedup vs baseline: 41.3581x; 1.0401x over previous
import jax
import jax.numpy as jnp
from jax.experimental import pallas as pl

_TOP_P = 0.8
_E = 64
_T_BLK = 1024


def _gate_kernel(x_ref, w_ref, o_ref):
    x = x_ref[...]
    w = w_ref[...]
    logits_t = jax.lax.dot_general(
        w, x, (((1,), (1,)), ((), ())),
        preferred_element_type=jnp.float32,
    )
    m = jnp.max(logits_t, axis=0, keepdims=True)
    ex = jnp.exp(logits_t - m)
    probs = ex / jnp.sum(ex, axis=0, keepdims=True)

    row = jax.lax.broadcasted_iota(jnp.int32, probs.shape, 0)
    rows = []
    for e in range(_E):
        pe = probs[e:e + 1, :]
        above = (probs > pe) | ((probs == pe) & (row < e))
        s_e = jnp.sum(jnp.where(above, probs, 0.0), axis=0, keepdims=True)
        rows.append(s_e)
    s = jnp.concatenate(rows, axis=0)
    out_t = jnp.where(s <= _TOP_P, 1.0, 0.0)
    eye = (jax.lax.broadcasted_iota(jnp.int32, (_E, _E), 0)
           == jax.lax.broadcasted_iota(jnp.int32, (_E, _E), 1)
           ).astype(jnp.float32)
    o_ref[...] = jax.lax.dot_general(
        out_t, eye, (((0,), (0,)), ((), ())),
        preferred_element_type=jnp.float32,
    )


def kernel(routing_inputs, W):
    n_tok, hidden = routing_inputs.shape
    return pl.pallas_call(
        _gate_kernel,
        grid=(n_tok // _T_BLK,),
        in_specs=[
            pl.BlockSpec((_T_BLK, hidden), lambda i: (i, 0)),
            pl.BlockSpec((_E, hidden), lambda i: (0, 0)),
        ],
        out_specs=pl.BlockSpec((_T_BLK, _E), lambda i: (i, 0)),
        out_shape=jax.ShapeDtypeStruct((n_tok, _E), jnp.float32),
    )(routing_inputs, W)
